# baseline (speedup 1.0000x reference)
import functools

import jax
import jax.numpy as jnp
from jax import lax
from jax.experimental import pallas as pl
from jax.experimental.pallas import tpu as pltpu
from jax.experimental.pallas import tpu_sc as plsc

VOCAB = 1000000
EMB_DIM = 64
MASK_DIM = 50
B = 4096
L = 200
OUT_DIM = EMB_DIM + MASK_DIM

NC, NS = 2, 16
NW = NC * NS
TI = L // 8
TJ = B // 128
NSP = 4
NU = TI * NSP
UROWS = 2 * 128


def _worker_id():
    return lax.axis_index("s") * NC + lax.axis_index("c")


def _gather_idx(sidx, b, s):
    return sidx.at[b, s]


def _scatter(ref, idxs, val):
    plsc.store_scatter(ref, idxs, val)


def _bcast_gather(ref, idxs):
    return plsc.load_gather(ref, idxs)


def _body(sent_ref, mask_ref, word_ref, mt_ref, out_ref,
          sidx, mbuf, wbuf, vbuf, mt_v, gsem0, gsem1, osem0, osem1):
    w = _worker_id()
    gsem = (gsem0, gsem1)
    osem = (osem0, osem1)

    pltpu.sync_copy(mt_ref.at[0], mt_v.at[pl.ds(0, MASK_DIM)])
    pltpu.sync_copy(mt_ref.at[1], mt_v.at[pl.ds(64, MASK_DIM)])
    b0 = [lax.bitcast_convert_type(mt_v[pl.ds(o, 16)], jnp.int32)
          for o in (0, 16, 32, 34)]
    b1 = [lax.bitcast_convert_type(mt_v[pl.ds(o, 16)], jnp.int32)
          for o in (64, 80, 96, 98)]
    bx = [a ^ b for a, b in zip(b0, b1)]
    iota = lax.iota(jnp.int32, 16)
    dw = [iota + 16 * k for k in range(4)]
    dm = [iota + 64 + o for o in (0, 16, 32, 34)]

    def fire_gathers(u, b):
        i, sp = u // NSP, u % NSP
        pltpu.sync_copy(sent_ref.at[i, w, pl.ds(2 * sp, 2)], sidx.at[b])
        for s in range(2):
            pltpu.async_copy(word_ref.at[_gather_idx(sidx, b, s)],
                             wbuf.at[b, pl.ds(s * 128, 128)], gsem[b])

    def do_unit(u, b, first, last):
        i, sp = u // NSP, u % NSP

        if not last:
            fire_gathers(u + 1, 1 - b)

        pltpu.make_async_copy(word_ref.at[pl.ds(0, UROWS)], wbuf.at[b],
                              gsem[b]).wait()
        for s in range(2):
            pltpu.sync_copy(mask_ref.at[i, w, 2 * sp + s],
                            mbuf.at[b, pl.ds(s * 128, 128)])

        @pl.when(jnp.logical_not(first))
        def _():
            pltpu.make_async_copy(vbuf.at[b],
                                  out_ref.at[:, 0, 0, pl.ds(0, 2)],
                                  osem[b]).wait()

        vb = vbuf.at[b]
        wb = wbuf.at[b]
        mb = mbuf.at[b]

        for s in range(2):
            sv = jnp.full((16,), s, jnp.int32)

            def row_body(m, _):
                r = s * 128 + m
                mv_full = jnp.full((16,), m, jnp.int32)
                for k in range(4):
                    _scatter(vb, [dw[k], sv, mv_full],
                             wb[r, pl.ds(16 * k, 16)])
                neg = -_bcast_gather(mbuf, [jnp.full((16,), b, jnp.int32),
                                            jnp.full((16,), r, jnp.int32)])
                for k in range(4):
                    sel = lax.bitcast_convert_type(b0[k] ^ (bx[k] & neg),
                                                   jnp.float32)
                    _scatter(vb, [dm[k], sv, mv_full], sel)
                return 0

            lax.fori_loop(0, 128, row_body, 0)

        pltpu.async_copy(vb, out_ref.at[:, i, w, pl.ds(2 * sp, 2)], osem[b])

    fire_gathers(0, 0)

    def pair_body(k, _):
        do_unit(2 * k, 0, k == 0, False)

        @pl.when(k < NU // 2 - 1)
        def _():
            do_unit(2 * k + 1, 1, k == 0, False)
        return 0

    lax.fori_loop(0, NU // 2, pair_body, 0)
    do_unit(NU - 1, 1, False, True)

    for b in range(2):
        pltpu.make_async_copy(vbuf.at[b], out_ref.at[:, 0, 0, pl.ds(0, 2)],
                              osem[b]).wait()


@jax.jit
def _run(sent4d, mask4d, word_table, mask_table):
    mesh = plsc.VectorSubcoreMesh(core_axis_name="c", subcore_axis_name="s")
    k = pl.kernel(
        _body,
        out_type=jax.ShapeDtypeStruct((OUT_DIM, TI, TJ, 8, 128), jnp.float32),
        mesh=mesh,
        compiler_params=pltpu.CompilerParams(use_tc_tiling_on_sc=False,
                                             needs_layout_passes=False),
        scratch_types=[
            pltpu.VMEM((2, 2, 128), jnp.int32),
            pltpu.VMEM((2, UROWS), jnp.int32),
            pltpu.VMEM((2, UROWS, EMB_DIM), jnp.float32),
            pltpu.VMEM((2, OUT_DIM, 2, 128), jnp.float32),
            pltpu.VMEM((128,), jnp.float32),
            pltpu.SemaphoreType.DMA,
            pltpu.SemaphoreType.DMA,
            pltpu.SemaphoreType.DMA,
            pltpu.SemaphoreType.DMA,
        ],
    )
    return k(sent4d, mask4d, word_table, mask_table)


def _tile_view(x):
    return jnp.transpose(jnp.transpose(x).reshape(TI, 8, TJ, 128),
                         (0, 2, 1, 3))


def kernel(sent, mask, word_table, mask_table):
    out5d = _run(_tile_view(sent), _tile_view(mask), word_table, mask_table)
    x = jnp.transpose(out5d, (1, 3, 2, 4, 0)).reshape(L, B, OUT_DIM)
    return jnp.transpose(x, (1, 0, 2))


# speedup vs baseline: 1.1563x; 1.1563x over previous
import functools

import jax
import jax.numpy as jnp
from jax import lax
from jax.experimental import pallas as pl
from jax.experimental.pallas import tpu as pltpu
from jax.experimental.pallas import tpu_sc as plsc

VOCAB = 1000000
EMB_DIM = 64
MASK_DIM = 50
B = 4096
L = 200
OUT_DIM = EMB_DIM + MASK_DIM
WPAD = EMB_DIM

NC, NS = 2, 16
NW = NC * NS
TI = L // 8
TJ = B // 128
NSP = 4
NU = TI * NSP
UROWS = 2 * 128


def _worker_id():
    return lax.axis_index("s") * NC + lax.axis_index("c")


def _gather_idx(sidx, b, s):
    return sidx.at[b, s]


def _bcast_gather(ref, idxs):
    return plsc.load_gather(ref, idxs)


def _body(sent_ref, mask_ref, word_ref, mt_ref, out_ref,
          sidx, mbuf, wbuf, vbuf, mt_v, trep, gsem0, gsem1, osem0, osem1):
    w = _worker_id()
    gsem = (gsem0, gsem1)
    osem = (osem0, osem1)

    pltpu.sync_copy(mt_ref.at[0], mt_v.at[pl.ds(0, MASK_DIM)])
    pltpu.sync_copy(mt_ref.at[1], mt_v.at[pl.ds(64, MASK_DIM)])
    iota = lax.iota(jnp.int32, 16)

    def prep_d(d, _):
        t0 = lax.bitcast_convert_type(
            _bcast_gather(mt_v, [jnp.full((16,), d, jnp.int32)]), jnp.int32)
        t1 = lax.bitcast_convert_type(
            _bcast_gather(mt_v, [jnp.full((16,), 64 + d, jnp.int32)]),
            jnp.int32)
        trep[0, d] = t0
        trep[1, d] = t0 ^ t1
        return 0

    lax.fori_loop(0, MASK_DIM, prep_d, 0)

    def fire_gathers(u, b):
        i, sp = u // NSP, u % NSP
        pltpu.sync_copy(sent_ref.at[i, w, pl.ds(2 * sp, 2)], sidx.at[b])
        for s in range(2):
            pltpu.async_copy(word_ref.at[_gather_idx(sidx, b, s)],
                             wbuf.at[b, pl.ds(s * 128, 128)], gsem[b])

    def do_unit(u, b, first, last):
        i, sp = u // NSP, u % NSP

        if not last:
            fire_gathers(u + 1, 1 - b)

        pltpu.make_async_copy(word_ref.at[pl.ds(0, UROWS)], wbuf.at[b],
                              gsem[b]).wait()
        for s in range(2):
            pltpu.sync_copy(mask_ref.at[i, w, 2 * sp + s],
                            mbuf.at[b, pl.ds(s * 128, 128)])

        @pl.when(jnp.logical_not(first))
        def _():
            pltpu.make_async_copy(vbuf.at[b],
                                  out_ref.at[:, 0, 0, pl.ds(0, UROWS)],
                                  osem[b]).wait()

        vb = vbuf.at[b]
        wb = wbuf.at[b]
        mb = mbuf.at[b]

        def group_body(g, _):
            rv = iota + g * 16
            negm = -mb[pl.ds(g * 16, 16)]
            for d in range(EMB_DIM):
                val = _bcast_gather(wb, [rv, jnp.full((16,), d, jnp.int32)])
                vb[d, pl.ds(g * 16, 16)] = val
            for d in range(MASK_DIM):
                sel = lax.bitcast_convert_type(
                    trep[0, d] ^ (trep[1, d] & negm), jnp.float32)
                vb[EMB_DIM + d, pl.ds(g * 16, 16)] = sel
            return 0

        lax.fori_loop(0, UROWS // 16, group_body, 0)

        pltpu.async_copy(vb, out_ref.at[:, i, w, pl.ds(sp * UROWS, UROWS)],
                         osem[b])

    fire_gathers(0, 0)

    def pair_body(k, _):
        do_unit(2 * k, 0, k == 0, False)

        @pl.when(k < NU // 2 - 1)
        def _():
            do_unit(2 * k + 1, 1, k == 0, False)
        return 0

    lax.fori_loop(0, NU // 2, pair_body, 0)
    do_unit(NU - 1, 1, False, True)

    for b in range(2):
        pltpu.make_async_copy(vbuf.at[b],
                              out_ref.at[:, 0, 0, pl.ds(0, UROWS)],
                              osem[b]).wait()


@jax.jit
def _run(sent4d, mask4d, word65, mask_table):
    mesh = plsc.VectorSubcoreMesh(core_axis_name="c", subcore_axis_name="s")
    k = pl.kernel(
        _body,
        out_type=jax.ShapeDtypeStruct((OUT_DIM, TI, TJ, 8 * 128), jnp.float32),
        mesh=mesh,
        compiler_params=pltpu.CompilerParams(use_tc_tiling_on_sc=False,
                                             needs_layout_passes=False),
        scratch_types=[
            pltpu.VMEM((2, 2, 128), jnp.int32),
            pltpu.VMEM((2, UROWS), jnp.int32),
            pltpu.VMEM((2, UROWS, WPAD), jnp.float32),
            pltpu.VMEM((2, OUT_DIM, UROWS), jnp.float32),
            pltpu.VMEM((128,), jnp.float32),
            pltpu.VMEM((2, MASK_DIM, 16), jnp.int32),
            pltpu.SemaphoreType.DMA,
            pltpu.SemaphoreType.DMA,
            pltpu.SemaphoreType.DMA,
            pltpu.SemaphoreType.DMA,
        ],
    )
    return k(sent4d, mask4d, word65, mask_table)


def _tile_view(x):
    return jnp.transpose(jnp.transpose(x).reshape(TI, 8, TJ, 128),
                         (0, 2, 1, 3))


def kernel(sent, mask, word_table, mask_table):
    out4d = _run(_tile_view(sent), _tile_view(mask), word_table, mask_table)
    out5d = out4d.reshape(OUT_DIM, TI, TJ, 8, 128)
    x = jnp.transpose(out5d, (1, 3, 2, 4, 0)).reshape(L, B, OUT_DIM)
    return jnp.transpose(x, (1, 0, 2))


# speedup vs baseline: 1.4777x; 1.2779x over previous
import functools

import jax
import jax.numpy as jnp
from jax import lax
from jax.experimental import pallas as pl
from jax.experimental.pallas import tpu as pltpu
from jax.experimental.pallas import tpu_sc as plsc

VOCAB = 1000000
EMB_DIM = 64
MASK_DIM = 50
B = 4096
L = 200
OUT_DIM = EMB_DIM + MASK_DIM
WPAD = EMB_DIM

NC, NS = 2, 16
NW = NC * NS
TI = L // 8
TJ = B // 128
NSP = 4
NU = TI * NSP
UROWS = 2 * 128


def _worker_id():
    return lax.axis_index("s") * NC + lax.axis_index("c")


def _gather_idx(sidx, b, s):
    return sidx.at[b, s]


def _bcast_gather(ref, idxs):
    return plsc.load_gather(ref, idxs)


def _body(sent_ref, mask_ref, word_ref, mt_ref, out_ref,
          sidx, mbuf, wbuf, vbuf, mt_v, trep, w65,
          gsem0, gsem1, osem0, osem1):
    w = _worker_id()
    gsem = (gsem0, gsem1)
    osem = (osem0, osem1)
    i65 = lax.iota(jnp.int32, 16) * 65

    pltpu.sync_copy(mt_ref.at[0], mt_v.at[pl.ds(0, MASK_DIM)])
    pltpu.sync_copy(mt_ref.at[1], mt_v.at[pl.ds(64, MASK_DIM)])
    iota = lax.iota(jnp.int32, 16)

    def prep_d(d, _):
        t0 = lax.bitcast_convert_type(
            _bcast_gather(mt_v, [jnp.full((16,), d, jnp.int32)]), jnp.int32)
        t1 = lax.bitcast_convert_type(
            _bcast_gather(mt_v, [jnp.full((16,), 64 + d, jnp.int32)]),
            jnp.int32)
        trep[0, d] = t0
        trep[1, d] = t0 ^ t1
        return 0

    lax.fori_loop(0, MASK_DIM, prep_d, 0)

    def fire_gathers(u, b):
        i, sp = u // NSP, u % NSP
        pltpu.sync_copy(sent_ref.at[i, w, pl.ds(2 * sp, 2)], sidx.at[b])
        for s in range(2):
            pltpu.async_copy(word_ref.at[_gather_idx(sidx, b, s)],
                             wbuf.at[b, pl.ds(s * 128, 128)], gsem[b])

    def do_unit(u, b, first, last):
        i, sp = u // NSP, u % NSP

        if not last:
            fire_gathers(u + 1, 1 - b)

        pltpu.make_async_copy(word_ref.at[pl.ds(0, UROWS)], wbuf.at[b],
                              gsem[b]).wait()
        for s in range(2):
            pltpu.sync_copy(mask_ref.at[i, w, 2 * sp + s],
                            mbuf.at[b, pl.ds(s * 128, 128)])

        @pl.when(jnp.logical_not(first))
        def _():
            pltpu.make_async_copy(vbuf.at[b],
                                  out_ref.at[:, 0, 0, pl.ds(0, UROWS)],
                                  osem[b]).wait()

        vb = vbuf.at[b]
        wb = wbuf.at[b]
        mb = mbuf.at[b]

        def group_body(g, _):
            base = g * 16
            for l in range(16):
                for k in range(EMB_DIM // 16):
                    w65[pl.ds(l * 65 + 16 * k, 16)] = \
                        wb[base + l, pl.ds(16 * k, 16)]
            negm = -mb[pl.ds(base, 16)]
            for d in range(EMB_DIM):
                val = _bcast_gather(w65, [i65 + d])
                vb[d, pl.ds(base, 16)] = val
            for d in range(MASK_DIM):
                sel = lax.bitcast_convert_type(
                    trep[0, d] ^ (trep[1, d] & negm), jnp.float32)
                vb[EMB_DIM + d, pl.ds(base, 16)] = sel
            return 0

        lax.fori_loop(0, UROWS // 16, group_body, 0)

        pltpu.async_copy(vb, out_ref.at[:, i, w, pl.ds(sp * UROWS, UROWS)],
                         osem[b])

    fire_gathers(0, 0)

    def pair_body(k, _):
        do_unit(2 * k, 0, k == 0, False)

        @pl.when(k < NU // 2 - 1)
        def _():
            do_unit(2 * k + 1, 1, k == 0, False)
        return 0

    lax.fori_loop(0, NU // 2, pair_body, 0)
    do_unit(NU - 1, 1, False, True)

    for b in range(2):
        pltpu.make_async_copy(vbuf.at[b],
                              out_ref.at[:, 0, 0, pl.ds(0, UROWS)],
                              osem[b]).wait()


@jax.jit
def _run(sent4d, mask4d, word65, mask_table):
    mesh = plsc.VectorSubcoreMesh(core_axis_name="c", subcore_axis_name="s")
    k = pl.kernel(
        _body,
        out_type=jax.ShapeDtypeStruct((OUT_DIM, TI, TJ, 8 * 128), jnp.float32),
        mesh=mesh,
        compiler_params=pltpu.CompilerParams(use_tc_tiling_on_sc=False,
                                             needs_layout_passes=False),
        scratch_types=[
            pltpu.VMEM((2, 2, 128), jnp.int32),
            pltpu.VMEM((2, UROWS), jnp.int32),
            pltpu.VMEM((2, UROWS, WPAD), jnp.float32),
            pltpu.VMEM((2, OUT_DIM, UROWS), jnp.float32),
            pltpu.VMEM((128,), jnp.float32),
            pltpu.VMEM((2, MASK_DIM, 16), jnp.int32),
            pltpu.VMEM((16 * 65,), jnp.float32),
            pltpu.SemaphoreType.DMA,
            pltpu.SemaphoreType.DMA,
            pltpu.SemaphoreType.DMA,
            pltpu.SemaphoreType.DMA,
        ],
    )
    return k(sent4d, mask4d, word65, mask_table)


def _tile_view(x):
    return jnp.transpose(jnp.transpose(x).reshape(TI, 8, TJ, 128),
                         (0, 2, 1, 3))


def kernel(sent, mask, word_table, mask_table):
    out4d = _run(_tile_view(sent), _tile_view(mask), word_table, mask_table)
    out5d = out4d.reshape(OUT_DIM, TI, TJ, 8, 128)
    x = jnp.transpose(out5d, (1, 3, 2, 4, 0)).reshape(L, B, OUT_DIM)
    return jnp.transpose(x, (1, 0, 2))
